# baseline (speedup 1.0000x reference)
import functools

import jax
import jax.numpy as jnp
from jax import lax
from jax.experimental import pallas as pl
from jax.experimental.pallas import tpu as pltpu
from jax.experimental.pallas import tpu_sc as plsc

_L = 1_000_000
_B = 2_000
_NB = _L // _B
_NR = 32


def _dist_body(recv_ref, blk_ref, big_ref, idx_ref, cx_ref, cy_ref, cz_ref,
               bv_ref, bi_ref, bx_ref, by_ref, bz_ref):
    step = pl.program_id(0)
    blk = blk_ref[...]
    x = blk[:, 0:1]
    y = blk[:, 1:2]
    z = blk[:, 2:3]
    rx = recv_ref[0:1, :]
    ry = recv_ref[1:2, :]
    rz = recv_ref[2:3, :]
    dx = x - rx
    dy = y - ry
    dz = z - rz
    d2 = dx * dx + dy * dy + dz * dz
    m = jnp.min(d2, axis=0, keepdims=True)
    rid = lax.broadcasted_iota(jnp.int32, (_B, _NR), 0) + step * _B
    cand = jnp.min(jnp.where(d2 == m, rid, _L), axis=0, keepdims=True)
    sel = d2 == m
    selc = rid == cand
    sel = jnp.logical_and(sel, selc)
    zero = jnp.zeros((), jnp.float32)
    wx = jnp.sum(jnp.where(sel, jnp.broadcast_to(x, (_B, _NR)), zero),
                 axis=0, keepdims=True)
    wy = jnp.sum(jnp.where(sel, jnp.broadcast_to(y, (_B, _NR)), zero),
                 axis=0, keepdims=True)
    wz = jnp.sum(jnp.where(sel, jnp.broadcast_to(z, (_B, _NR)), zero),
                 axis=0, keepdims=True)

    @pl.when(step == 0)
    def _():
        bv_ref[...] = jnp.full((1, _NR), jnp.inf, jnp.float32)
        bi_ref[...] = jnp.zeros((1, _NR), jnp.int32)
        bx_ref[...] = jnp.zeros((1, _NR), jnp.float32)
        by_ref[...] = jnp.zeros((1, _NR), jnp.float32)
        bz_ref[...] = jnp.zeros((1, _NR), jnp.float32)

    upd = m < bv_ref[...]
    bv_ref[...] = jnp.where(upd, m, bv_ref[...])
    bi_ref[...] = jnp.where(upd, cand, bi_ref[...])
    bx_ref[...] = jnp.where(upd, wx, bx_ref[...])
    by_ref[...] = jnp.where(upd, wy, by_ref[...])
    bz_ref[...] = jnp.where(upd, wz, bz_ref[...])

    big_ref[:, 0:3] = blk
    big_ref[:, 3:4] = jnp.zeros((_B, 1), jnp.float32)

    @pl.when(step == _NB - 1)
    def _():
        idx_ref[...] = jnp.broadcast_to(bi_ref[...], (8, _NR))
        cx_ref[...] = jnp.broadcast_to(bx_ref[...], (8, _NR))
        cy_ref[...] = jnp.broadcast_to(by_ref[...], (8, _NR))
        cz_ref[...] = jnp.broadcast_to(bz_ref[...], (8, _NR))


_dist_call = pl.pallas_call(
    _dist_body,
    grid=(_NB,),
    in_specs=[
        pl.BlockSpec((8, _NR), lambda i: (0, 0)),
        pl.BlockSpec((_B, 3), lambda i: (i, 0)),
    ],
    out_specs=[
        pl.BlockSpec((_B, 4), lambda i: (i, 0)),
        pl.BlockSpec((8, _NR), lambda i: (0, 0)),
        pl.BlockSpec((8, _NR), lambda i: (0, 0)),
        pl.BlockSpec((8, _NR), lambda i: (0, 0)),
        pl.BlockSpec((8, _NR), lambda i: (0, 0)),
    ],
    out_shape=[
        jax.ShapeDtypeStruct((_L, 4), jnp.float32),
        jax.ShapeDtypeStruct((8, _NR), jnp.int32),
        jax.ShapeDtypeStruct((8, _NR), jnp.float32),
        jax.ShapeDtypeStruct((8, _NR), jnp.float32),
        jax.ShapeDtypeStruct((8, _NR), jnp.float32),
    ],
    scratch_shapes=[
        pltpu.VMEM((1, _NR), jnp.float32),
        pltpu.VMEM((1, _NR), jnp.int32),
        pltpu.VMEM((1, _NR), jnp.float32),
        pltpu.VMEM((1, _NR), jnp.float32),
        pltpu.VMEM((1, _NR), jnp.float32),
    ],
)


def _sc_scatter_body(idx_hbm, rows_hbm, big_ref, idx_v, rows_v, sem):
    wid = lax.axis_index("c") * 16 + lax.axis_index("s")

    @pl.when(wid == 0)
    def _():
        pltpu.sync_copy(idx_hbm, idx_v)
        pltpu.sync_copy(rows_hbm, rows_v)
        vecs = [idx_v[pl.ds(0, 16)], idx_v[pl.ds(16, 16)]]
        copies = []
        for j in range(_NR):
            row = vecs[j // 16][j % 16]
            copies.append(pltpu.async_copy(
                rows_v.at[pl.ds(j, 1), :],
                big_ref.at[pl.ds(row, 1), :],
                sem))
        for c in copies:
            c.wait()


@functools.lru_cache(maxsize=None)
def _make_sc_scatter():
    mesh = plsc.VectorSubcoreMesh(core_axis_name="c", subcore_axis_name="s")
    return pl.kernel(
        _sc_scatter_body,
        out_type=(),
        mesh=mesh,
        scratch_types=[
            pltpu.VMEM((_NR,), jnp.int32),
            pltpu.VMEM((_NR, 4), jnp.float32),
            pltpu.SemaphoreType.DMA,
        ],
    )


def kernel(mesh_3D, receiver_pos):
    mesh_flat = mesh_3D.reshape(_L, 3)
    recv = jnp.zeros((8, _NR), jnp.float32).at[0:3, :].set(
        receiver_pos.T.astype(jnp.float32))
    big, idx8, cx8, cy8, cz8 = _dist_call(recv, mesh_flat)
    idx = idx8[0].astype(jnp.int32)
    closest = jnp.stack([cx8[0], cy8[0], cz8[0]], axis=1)
    rows = jnp.concatenate(
        [closest, jnp.ones((_NR, 1), jnp.float32)], axis=1)
    big_ref = jax.new_ref(big)
    _make_sc_scatter()(idx, rows, big_ref)
    return big_ref[...], closest
